# baseline (speedup 1.0000x reference)
import jax
import jax.numpy as jnp
from jax.experimental import pallas as pl
from jax.experimental.pallas import tpu as pltpu

_L = 2048
_D = 512
_G = 3 * _D
_PAD = 8
_C = 256


def _localrnn_kernel(x_ref, wih_ref, whh_ref, bih_ref, bhh_ref, o_ref, g_s):
    bih = bih_ref[...]
    bhh = bhh_ref[...]
    wih = wih_ref[...]
    whh = whh_ref[...]

    g_s[0:_PAD, :] = jnp.broadcast_to(bih, (_PAD, _G))
    g_s[_PAD:, :] = jnp.dot(x_ref[0], wih,
                            preferred_element_type=jnp.float32) + bih

    bhh_r = bhh[:, 0:_D]
    bhh_z = bhh[:, _D:2 * _D]
    bhh_n = bhh[:, 2 * _D:]

    n_chunks = _L // _C
    for ci in range(n_chunks):
        c0 = ci * _C

        g0 = g_s[c0 + _PAD - 2:c0 + _PAD - 2 + _C, :]
        r = jax.nn.sigmoid(g0[:, 0:_D] + bhh_r)
        z = jax.nn.sigmoid(g0[:, _D:2 * _D] + bhh_z)
        n = jnp.tanh(g0[:, 2 * _D:] + r * bhh_n)
        h = n - z * n

        for t in (1, 2):
            g = g_s[c0 + _PAD - 2 + t:c0 + _PAD - 2 + t + _C, :]
            gh = jnp.dot(h.astype(jnp.bfloat16), whh,
                         preferred_element_type=jnp.float32) + bhh
            r = jax.nn.sigmoid(g[:, 0:_D] + gh[:, 0:_D])
            z = jax.nn.sigmoid(g[:, _D:2 * _D] + gh[:, _D:2 * _D])
            n = jnp.tanh(g[:, 2 * _D:] + r * gh[:, 2 * _D:])
            h = n + z * (h - n)

        o_ref[0, c0:c0 + _C, :] = h


@jax.jit
def kernel(x, W_ih, W_hh, b_ih, b_hh):
    B, L, D = x.shape
    wih_t = W_ih.T.astype(jnp.bfloat16)
    whh_t = W_hh.T.astype(jnp.bfloat16)
    bih2 = b_ih.reshape(1, _G)
    bhh2 = b_hh.reshape(1, _G)
    xb = x.astype(jnp.bfloat16)

    return pl.pallas_call(
        _localrnn_kernel,
        out_shape=jax.ShapeDtypeStruct((B, L, D), x.dtype),
        grid=(B,),
        in_specs=[
            pl.BlockSpec((1, L, D), lambda b: (b, 0, 0)),
            pl.BlockSpec((D, _G), lambda b: (0, 0)),
            pl.BlockSpec((D, _G), lambda b: (0, 0)),
            pl.BlockSpec((1, _G), lambda b: (0, 0)),
            pl.BlockSpec((1, _G), lambda b: (0, 0)),
        ],
        out_specs=pl.BlockSpec((1, L, D), lambda b: (b, 0, 0)),
        scratch_shapes=[pltpu.VMEM((_L + _PAD, _G), jnp.float32)],
        compiler_params=pltpu.CompilerParams(
            dimension_semantics=("parallel",),
            vmem_limit_bytes=56 * 1024 * 1024,
        ),
        name="localrnn_gru3",
    )(xb, wih_t, whh_t, bih2, bhh2)


# speedup vs baseline: 1.1058x; 1.1058x over previous
import jax
import jax.numpy as jnp
from jax.experimental import pallas as pl
from jax.experimental.pallas import tpu as pltpu

_L = 2048
_D = 512
_G = 3 * _D
_PAD = 8
_C = 256


def _localrnn_kernel(x_ref, wih_ref, whh_ref, bih_ref, bhh_ref, o_ref, g_s):
    x = x_ref[0]
    bih = bih_ref[...]
    bhh = bhh_ref[...]

    g_s[0:_PAD, :] = jnp.broadcast_to(bih, (_PAD, _G))
    g_s[_PAD:, :] = jnp.dot(x, wih_ref[...], preferred_element_type=jnp.float32) + bih

    bhh_r = bhh[:, 0:_D]
    bhh_z = bhh[:, _D:2 * _D]
    bhh_n = bhh[:, 2 * _D:]

    for c0 in range(0, _L, _C):
        g0 = g_s[c0 + _PAD - 2:c0 + _PAD - 2 + _C, :]
        r = jax.nn.sigmoid(g0[:, 0:_D] + bhh_r)
        z = jax.nn.sigmoid(g0[:, _D:2 * _D] + bhh_z)
        n = jnp.tanh(g0[:, 2 * _D:] + r * bhh_n)
        h = n - z * n

        for t in (1, 2):
            g = g_s[c0 + _PAD - 2 + t:c0 + _PAD - 2 + t + _C, :]
            gh = jnp.dot(h, whh_ref[...], preferred_element_type=jnp.float32) + bhh
            r = jax.nn.sigmoid(g[:, 0:_D] + gh[:, 0:_D])
            z = jax.nn.sigmoid(g[:, _D:2 * _D] + gh[:, _D:2 * _D])
            n = jnp.tanh(g[:, 2 * _D:] + r * gh[:, 2 * _D:])
            h = n + z * (h - n)

        o_ref[0, c0:c0 + _C, :] = h


@jax.jit
def kernel(x, W_ih, W_hh, b_ih, b_hh):
    B, L, D = x.shape
    wih_t = W_ih.T
    whh_t = W_hh.T
    bih2 = b_ih.reshape(1, _G)
    bhh2 = b_hh.reshape(1, _G)

    return pl.pallas_call(
        _localrnn_kernel,
        out_shape=jax.ShapeDtypeStruct((B, L, D), x.dtype),
        grid=(B,),
        in_specs=[
            pl.BlockSpec((1, L, D), lambda b: (b, 0, 0)),
            pl.BlockSpec((D, _G), lambda b: (0, 0)),
            pl.BlockSpec((D, _G), lambda b: (0, 0)),
            pl.BlockSpec((1, _G), lambda b: (0, 0)),
            pl.BlockSpec((1, _G), lambda b: (0, 0)),
        ],
        out_specs=pl.BlockSpec((1, L, D), lambda b: (b, 0, 0)),
        scratch_shapes=[pltpu.VMEM((_L + _PAD, _G), jnp.float32)],
        compiler_params=pltpu.CompilerParams(
            dimension_semantics=("parallel",),
            vmem_limit_bytes=56 * 1024 * 1024,
        ),
        name="localrnn_gru3",
    )(x, wih_t, whh_t, bih2, bhh2)


# speedup vs baseline: 1.1174x; 1.0104x over previous
import jax
import jax.numpy as jnp
from jax.experimental import pallas as pl
from jax.experimental.pallas import tpu as pltpu

_L = 2048
_D = 512
_G = 3 * _D
_PAD = 8
_C = 256
_S = 64


def _localrnn_kernel(x_ref, wih_ref, whh_ref, bih_ref, bhh_ref, o_ref, g_s):
    x = x_ref[0]
    bih = bih_ref[...]
    bhh = bhh_ref[...]

    g_s[0:_PAD, :] = jnp.broadcast_to(bih, (_PAD, _G))
    g_s[_PAD:, :] = jnp.dot(x, wih_ref[...], preferred_element_type=jnp.float32) + bih

    bhh_r = bhh[:, 0:_D]
    bhh_z = bhh[:, _D:2 * _D]
    bhh_n = bhh[:, 2 * _D:]

    for c0 in range(0, _L, _C):
        hs = []
        for s0 in range(0, _C, _S):
            g0 = g_s[c0 + _PAD - 2 + s0:c0 + _PAD - 2 + s0 + _S, :]
            r = jax.nn.sigmoid(g0[:, 0:_D] + bhh_r)
            z = jax.nn.sigmoid(g0[:, _D:2 * _D] + bhh_z)
            n = jnp.tanh(g0[:, 2 * _D:] + r * bhh_n)
            hs.append((1.0 - z) * n)
        h = jnp.concatenate(hs, axis=0)

        for t in (1, 2):
            gh = jnp.dot(h, whh_ref[...], preferred_element_type=jnp.float32) + bhh
            hs = []
            for s0 in range(0, _C, _S):
                g = g_s[c0 + _PAD - 2 + t + s0:c0 + _PAD - 2 + t + s0 + _S, :]
                ghs = gh[s0:s0 + _S]
                r = jax.nn.sigmoid(g[:, 0:_D] + ghs[:, 0:_D])
                z = jax.nn.sigmoid(g[:, _D:2 * _D] + ghs[:, _D:2 * _D])
                n = jnp.tanh(g[:, 2 * _D:] + r * ghs[:, 2 * _D:])
                hs.append((1.0 - z) * n + z * h[s0:s0 + _S])
            h = jnp.concatenate(hs, axis=0)

        o_ref[0, c0:c0 + _C, :] = h


@jax.jit
def kernel(x, W_ih, W_hh, b_ih, b_hh):
    B, L, D = x.shape
    wih_t = W_ih.T
    whh_t = W_hh.T
    bih2 = b_ih.reshape(1, _G)
    bhh2 = b_hh.reshape(1, _G)

    return pl.pallas_call(
        _localrnn_kernel,
        out_shape=jax.ShapeDtypeStruct((B, L, D), x.dtype),
        grid=(B,),
        in_specs=[
            pl.BlockSpec((1, L, D), lambda b: (b, 0, 0)),
            pl.BlockSpec((D, _G), lambda b: (0, 0)),
            pl.BlockSpec((D, _G), lambda b: (0, 0)),
            pl.BlockSpec((1, _G), lambda b: (0, 0)),
            pl.BlockSpec((1, _G), lambda b: (0, 0)),
        ],
        out_specs=pl.BlockSpec((1, L, D), lambda b: (b, 0, 0)),
        scratch_shapes=[pltpu.VMEM((_L + _PAD, _G), jnp.float32)],
        compiler_params=pltpu.CompilerParams(
            dimension_semantics=("parallel",),
            vmem_limit_bytes=56 * 1024 * 1024,
        ),
        name="localrnn_gru3",
    )(x, wih_t, whh_t, bih2, bhh2)


# speedup vs baseline: 1.2450x; 1.1142x over previous
import jax
import jax.numpy as jnp
from jax.experimental import pallas as pl
from jax.experimental.pallas import tpu as pltpu

_L = 2048
_D = 512
_G = 3 * _D
_PAD = 8
_C = 256


def _localrnn_kernel(x_ref, wih_ref, whh_ref, bih_ref, bhh_ref, o_ref, g_s):
    bih = bih_ref[...]
    bhh = bhh_ref[...]
    wih = wih_ref[...]
    whh = whh_ref[...]

    def proj(c0):
        g_s[_PAD + c0:_PAD + c0 + _C, :] = (
            jnp.dot(x_ref[0, c0:c0 + _C, :], wih,
                    preferred_element_type=jnp.float32) + bih)

    g_s[0:_PAD, :] = jnp.broadcast_to(bih, (_PAD, _G))
    proj(0)

    bhh_r = bhh[:, 0:_D]
    bhh_z = bhh[:, _D:2 * _D]
    bhh_n = bhh[:, 2 * _D:]

    n_chunks = _L // _C
    for ci in range(n_chunks):
        c0 = ci * _C
        if ci + 1 < n_chunks:
            proj(c0 + _C)

        g0 = g_s[c0 + _PAD - 2:c0 + _PAD - 2 + _C, :]
        r = jax.nn.sigmoid(g0[:, 0:_D] + bhh_r)
        z = jax.nn.sigmoid(g0[:, _D:2 * _D] + bhh_z)
        n = jnp.tanh(g0[:, 2 * _D:] + r * bhh_n)
        h = (1.0 - z) * n

        for t in (1, 2):
            g = g_s[c0 + _PAD - 2 + t:c0 + _PAD - 2 + t + _C, :]
            gh = jnp.dot(h, whh, preferred_element_type=jnp.float32) + bhh
            r = jax.nn.sigmoid(g[:, 0:_D] + gh[:, 0:_D])
            z = jax.nn.sigmoid(g[:, _D:2 * _D] + gh[:, _D:2 * _D])
            n = jnp.tanh(g[:, 2 * _D:] + r * gh[:, 2 * _D:])
            h = (1.0 - z) * n + z * h

        o_ref[0, c0:c0 + _C, :] = h


@jax.jit
def kernel(x, W_ih, W_hh, b_ih, b_hh):
    B, L, D = x.shape
    wih_t = W_ih.T
    whh_t = W_hh.T
    bih2 = b_ih.reshape(1, _G)
    bhh2 = b_hh.reshape(1, _G)

    return pl.pallas_call(
        _localrnn_kernel,
        out_shape=jax.ShapeDtypeStruct((B, L, D), x.dtype),
        grid=(B,),
        in_specs=[
            pl.BlockSpec((1, L, D), lambda b: (b, 0, 0)),
            pl.BlockSpec((D, _G), lambda b: (0, 0)),
            pl.BlockSpec((D, _G), lambda b: (0, 0)),
            pl.BlockSpec((1, _G), lambda b: (0, 0)),
            pl.BlockSpec((1, _G), lambda b: (0, 0)),
        ],
        out_specs=pl.BlockSpec((1, L, D), lambda b: (b, 0, 0)),
        scratch_shapes=[pltpu.VMEM((_L + _PAD, _G), jnp.float32)],
        compiler_params=pltpu.CompilerParams(
            dimension_semantics=("parallel",),
            vmem_limit_bytes=56 * 1024 * 1024,
        ),
        name="localrnn_gru3",
    )(x, wih_t, whh_t, bih2, bhh2)


# speedup vs baseline: 1.2824x; 1.0301x over previous
import jax
import jax.numpy as jnp
from jax.experimental import pallas as pl
from jax.experimental.pallas import tpu as pltpu

_L = 2048
_D = 512
_G = 3 * _D
_PAD = 8
_C = 512


def _localrnn_kernel(x_ref, wih_ref, whh_ref, bih_ref, bhh_ref, o_ref, g_s):
    bih = bih_ref[...]
    bhh = bhh_ref[...]
    wih = wih_ref[...]
    whh = whh_ref[...]

    def proj(c0):
        g_s[_PAD + c0:_PAD + c0 + _C, :] = (
            jnp.dot(x_ref[0, c0:c0 + _C, :], wih,
                    preferred_element_type=jnp.float32) + bih)

    g_s[0:_PAD, :] = jnp.broadcast_to(bih, (_PAD, _G))
    proj(0)

    bhh_r = bhh[:, 0:_D]
    bhh_z = bhh[:, _D:2 * _D]
    bhh_n = bhh[:, 2 * _D:]

    n_chunks = _L // _C
    for ci in range(n_chunks):
        c0 = ci * _C
        if ci + 1 < n_chunks:
            proj(c0 + _C)

        g0 = g_s[c0 + _PAD - 2:c0 + _PAD - 2 + _C, :]
        r = jax.nn.sigmoid(g0[:, 0:_D] + bhh_r)
        z = jax.nn.sigmoid(g0[:, _D:2 * _D] + bhh_z)
        n = jnp.tanh(g0[:, 2 * _D:] + r * bhh_n)
        h = (1.0 - z) * n

        for t in (1, 2):
            g = g_s[c0 + _PAD - 2 + t:c0 + _PAD - 2 + t + _C, :]
            gh = jnp.dot(h, whh, preferred_element_type=jnp.float32) + bhh
            r = jax.nn.sigmoid(g[:, 0:_D] + gh[:, 0:_D])
            z = jax.nn.sigmoid(g[:, _D:2 * _D] + gh[:, _D:2 * _D])
            n = jnp.tanh(g[:, 2 * _D:] + r * gh[:, 2 * _D:])
            h = (1.0 - z) * n + z * h

        o_ref[0, c0:c0 + _C, :] = h


@jax.jit
def kernel(x, W_ih, W_hh, b_ih, b_hh):
    B, L, D = x.shape
    wih_t = W_ih.T
    whh_t = W_hh.T
    bih2 = b_ih.reshape(1, _G)
    bhh2 = b_hh.reshape(1, _G)

    return pl.pallas_call(
        _localrnn_kernel,
        out_shape=jax.ShapeDtypeStruct((B, L, D), x.dtype),
        grid=(B,),
        in_specs=[
            pl.BlockSpec((1, L, D), lambda b: (b, 0, 0)),
            pl.BlockSpec((D, _G), lambda b: (0, 0)),
            pl.BlockSpec((D, _G), lambda b: (0, 0)),
            pl.BlockSpec((1, _G), lambda b: (0, 0)),
            pl.BlockSpec((1, _G), lambda b: (0, 0)),
        ],
        out_specs=pl.BlockSpec((1, L, D), lambda b: (b, 0, 0)),
        scratch_shapes=[pltpu.VMEM((_L + _PAD, _G), jnp.float32)],
        compiler_params=pltpu.CompilerParams(
            dimension_semantics=("parallel",),
            vmem_limit_bytes=56 * 1024 * 1024,
        ),
        name="localrnn_gru3",
    )(x, wih_t, whh_t, bih2, bhh2)
